# Initial kernel scaffold; baseline (speedup 1.0000x reference)
import jax
import jax.numpy as jnp
from jax.experimental import pallas as pl


def kernel(user_embedding, item_embedding, user_neg_embedding, item_neg_embedding, edge_index_p, edge_index_n):
    raise NotImplementedError("write your pallas kernel here")


# speedup vs baseline: 3.7649x; 3.7649x over previous
import functools

import jax
import jax.numpy as jnp
from jax import lax
from jax.experimental import pallas as pl
from jax.experimental.pallas import tpu as pltpu
from jax.experimental.pallas import tpu_sc as plsc

_M, _NV, _DIM = 2000, 8000, 256
_NN = _M + _NV
_E = 160000
_NC, _NS = 2, 16
_NPAD = 10240
_RPT = _NPAD // _NS
_CH = 64
_KCH = _RPT // _CH
_EPW = _E // _NS
_EPT = 10240
_ECH = _EPT // _CH
_HALF = 128
_NCOL = _HALF // 16
_SUB = _CH // 16
_SCW = _CH * 16
_ALPHA = 1.0 / 3.0


def _rsqrt_newton(x):
    y = jnp.full((16,), 0.70710678 * 2.0 ** -8, jnp.float32)
    for k in range(7, -1, -1):
        thr = jnp.full((16,), 4.0 ** (k + 1), jnp.float32)
        y = jnp.where(x < thr, jnp.full((16,), 0.70710678 * 2.0 ** -k, jnp.float32), y)
    c15 = jnp.full((16,), 1.5, jnp.float32)
    ch = jnp.full((16,), 0.5, jnp.float32)
    for _ in range(5):
        y = y * (c15 - ch * x * y * y)
    return y


def _body(e_pos, e_neg, sp_h, dp_h, sn_h, dn_h,
          pos_out, neg_out,
          t0, t1, t2, t3, p1h, n1h, tth, sc_h,
          svv, dvv,
          bufa, bufb,
          sb0, sb1, sb2, sb3,
          zbuf, onesb,
          acc,
          semg, sems):
    cid = lax.axis_index("c")
    sid = lax.axis_index("s")
    base = sid * _RPT
    coff = cid * _HALF
    toff = cid * _NPAD
    scb = (cid * 4) * _NPAD * 16

    def stage_idx(src_hbm, dst_hbm):
        pltpu.sync_copy(src_hbm.at[pl.ds(sid * _EPT, _EPT)], svv)
        pltpu.sync_copy(dst_hbm.at[pl.ds(sid * _EPT, _EPT)], dvv)

    def zfill(r, _):
        zv = jnp.zeros((16,), jnp.float32)
        for cc in range(_NCOL):
            zbuf[r, pl.ds(cc * 16, 16)] = zv
        return 0
    lax.fori_loop(0, 8, zfill, 0)

    def zero_acc_chunk(k):
        for q in range(_CH // 8):
            pltpu.sync_copy(zbuf, acc.at[pl.ds(base + k * _CH + q * 8, 8)])

    def zinit(k, _):
        zero_acc_chunk(k)
        return 0
    lax.fori_loop(0, _KCH, zinit, 0)
    plsc.subcore_barrier()

    for rnd, (s_hbm, d_hbm) in ((0, (sp_h, dp_h)), (1, (sn_h, dn_h))):
        stage_idx(s_hbm, d_hbm)
        for half, iv in ((0, svv), (1, dvv)):
            side = rnd * 2 + half
            def bandfill(r, _, side=side):
                for cc in range(_NCOL):
                    v = 1.0 if cc == side else 0.0
                    onesb[r, pl.ds(cc * 16, 16)] = jnp.full((16,), v, jnp.float32)
                return 0
            lax.fori_loop(0, 16, bandfill, 0)
            def dstep(g, _, iv=iv):
                for j in range(_SUB):
                    dv = iv[pl.ds(g * _CH + j * 16, 16)]
                    pltpu.async_copy(onesb.at[pl.ds(0, 16)], acc.at[dv],
                                     semg.at[0], add=True)
                for j in range(_SUB):
                    pltpu.make_async_copy(
                        onesb.at[pl.ds(0, 16)],
                        acc.at[jnp.zeros((16,), jnp.int32)],
                        semg.at[0]).wait()
                return 0
            lax.fori_loop(0, _ECH, dstep, 0)
    plsc.subcore_barrier()

    def scprod(k, _):
        pltpu.sync_copy(acc.at[pl.ds(base + k * _CH, _CH)], bufa)
        zero_acc_chunk(k)
        def abstep(r, _):
            for side, sref in ((0, sb0), (1, sb1), (2, sb2), (3, sb3)):
                d = jnp.maximum(bufa[r, pl.ds(side * 16, 16)],
                                jnp.full((16,), 1.0, jnp.float32))
                sref[pl.ds(r * 16, 16)] = _rsqrt_newton(d)
            return 0
        lax.fori_loop(0, _CH, abstep, 0)
        for side, sref in ((0, sb0), (1, sb1), (2, sb2), (3, sb3)):
            pltpu.sync_copy(
                sref,
                sc_h.at[pl.ds(scb + side * _NPAD * 16 + (base + k * _CH) * 16,
                              _SCW)])
        return 0
    lax.fori_loop(0, _KCH, scprod, 0)
    plsc.subcore_barrier()

    def fetch_scale(sref, side, k):
        pltpu.sync_copy(
            sc_h.at[pl.ds(scb + side * _NPAD * 16 + (base + k * _CH) * 16,
                          _SCW)], sref)

    def scale_to(dst_ref, src_ref, sref):
        def grp(r, _):
            sv = sref[pl.ds(r * 16, 16)]
            for cc in range(_NCOL):
                dst_ref[r, pl.ds(cc * 16, 16)] = src_ref[r, pl.ds(cc * 16, 16)] * sv
            return 0
        lax.fori_loop(0, _CH, grp, 0)

    def prescale_two(src_h, dsta_h, sidea, dstb_h, sideb):
        def pk(k, _):
            pltpu.sync_copy(
                src_h.at[pl.ds(base + k * _CH, _CH), pl.ds(coff, _HALF)], bufa)
            fetch_scale(sb0, sidea, k)
            fetch_scale(sb1, sideb, k)
            scale_to(bufb, bufa, sb0)
            pltpu.sync_copy(bufb, dsta_h.at[pl.ds(toff + base + k * _CH, _CH)])
            scale_to(bufb, bufa, sb1)
            pltpu.sync_copy(bufb, dstb_h.at[pl.ds(toff + base + k * _CH, _CH)])
            return 0
        lax.fori_loop(0, _KCH, pk, 0)

    def scatter_pass(tbl_h):
        def issue_gather(g, buf, par):
            for j in range(_SUB):
                iv = svv[pl.ds(g * _CH + j * 16, 16)] + jnp.full(
                    (16,), 1, jnp.int32) * toff
                pltpu.async_copy(tbl_h.at[iv], buf.at[pl.ds(j * 16, 16)],
                                 semg.at[par])

        def wait_gather(buf, par):
            for j in range(_SUB):
                pltpu.make_async_copy(tbl_h.at[jnp.zeros((16,), jnp.int32)],
                                      buf.at[pl.ds(j * 16, 16)],
                                      semg.at[par]).wait()

        def issue_scatter(g, buf, par):
            for j in range(_SUB):
                dv = dvv[pl.ds(g * _CH + j * 16, 16)]
                pltpu.async_copy(buf.at[pl.ds(j * 16, 16)], acc.at[dv],
                                 sems.at[par], add=True)

        def drain_scatter(buf, par):
            for j in range(_SUB):
                pltpu.make_async_copy(buf.at[pl.ds(j * 16, 16)],
                                      acc.at[jnp.zeros((16,), jnp.int32)],
                                      sems.at[par]).wait()

        issue_gather(0, bufa, 0)
        def pair(p, _):
            g2 = p * 2
            for par in range(2):
                g = g2 + par
                buf_cur = bufa if par == 0 else bufb
                buf_nxt = bufb if par == 0 else bufa
                wait_gather(buf_cur, par)
                issue_scatter(g, buf_cur, par)
                @pl.when(g + 1 < _ECH)
                def _():
                    @pl.when(g >= 1)
                    def _():
                        drain_scatter(buf_nxt, 1 - par)
                    issue_gather(g + 1, buf_nxt, 1 - par)
            return 0
        lax.fori_loop(0, _ECH // 2, pair, 0)
        drain_scatter(bufa, 0)
        drain_scatter(bufb, 1)

    def read_acc_scaled(k, side):
        pltpu.sync_copy(acc.at[pl.ds(base + k * _CH, _CH)], bufa)
        zero_acc_chunk(k)
        fetch_scale(sb0, side, k)
        scale_to(bufa, bufa, sb0)

    def add_from(src_h, k, coloff=None):
        if coloff is None:
            pltpu.sync_copy(src_h.at[pl.ds(toff + base + k * _CH, _CH)], bufb)
        else:
            pltpu.sync_copy(
                src_h.at[pl.ds(base + k * _CH, _CH), pl.ds(coloff, _HALF)], bufb)
        def grp(r, _):
            for cc in range(_NCOL):
                bufa[r, pl.ds(cc * 16, 16)] = (
                    bufa[r, pl.ds(cc * 16, 16)] + bufb[r, pl.ds(cc * 16, 16)])
            return 0
        lax.fori_loop(0, _CH, grp, 0)

    def scale_alpha():
        av = jnp.full((16,), _ALPHA, jnp.float32)
        def grp(r, _):
            for cc in range(_NCOL):
                bufa[r, pl.ds(cc * 16, 16)] = bufa[r, pl.ds(cc * 16, 16)] * av
            return 0
        lax.fori_loop(0, _CH, grp, 0)

    prescale_two(e_pos, t0, 0, t1, 2)
    plsc.subcore_barrier()

    stage_idx(sp_h, dp_h)
    scatter_pass(t0)
    plsc.subcore_barrier()
    def wb_l0p(k, _):
        read_acc_scaled(k, 1)
        pltpu.sync_copy(bufa, p1h.at[pl.ds(toff + base + k * _CH, _CH)])
        fetch_scale(sb1, 0, k)
        scale_to(bufb, bufa, sb1)
        pltpu.sync_copy(bufb, t0.at[pl.ds(toff + base + k * _CH, _CH)])
        fetch_scale(sb1, 2, k)
        scale_to(bufb, bufa, sb1)
        pltpu.sync_copy(bufb, t2.at[pl.ds(toff + base + k * _CH, _CH)])
        return 0
    lax.fori_loop(0, _KCH, wb_l0p, 0)
    plsc.subcore_barrier()

    stage_idx(sn_h, dn_h)
    scatter_pass(t1)
    plsc.subcore_barrier()
    def wb_l0n(k, _):
        read_acc_scaled(k, 3)
        pltpu.sync_copy(bufa, n1h.at[pl.ds(toff + base + k * _CH, _CH)])
        fetch_scale(sb1, 2, k)
        scale_to(bufb, bufa, sb1)
        pltpu.sync_copy(bufb, t1.at[pl.ds(toff + base + k * _CH, _CH)])
        fetch_scale(sb1, 0, k)
        scale_to(bufb, bufa, sb1)
        pltpu.sync_copy(bufb, t3.at[pl.ds(toff + base + k * _CH, _CH)])
        return 0
    lax.fori_loop(0, _KCH, wb_l0n, 0)
    plsc.subcore_barrier()

    stage_idx(sp_h, dp_h)
    scatter_pass(t0)
    plsc.subcore_barrier()
    def wb_t(k, _):
        read_acc_scaled(k, 1)
        pltpu.sync_copy(bufa, tth.at[pl.ds(toff + base + k * _CH, _CH)])
        return 0
    lax.fori_loop(0, _KCH, wb_t, 0)
    plsc.subcore_barrier()
    stage_idx(sn_h, dn_h)
    scatter_pass(t1)
    plsc.subcore_barrier()
    def wb_pos(k, _):
        read_acc_scaled(k, 3)
        add_from(tth, k)
        add_from(p1h, k)
        add_from(e_pos, k, coloff=coff)
        scale_alpha()
        pltpu.sync_copy(
            bufa, pos_out.at[pl.ds(base + k * _CH, _CH), pl.ds(coff, _HALF)])
        return 0
    lax.fori_loop(0, _KCH, wb_pos, 0)
    plsc.subcore_barrier()

    stage_idx(sp_h, dp_h)
    scatter_pass(t3)
    plsc.subcore_barrier()
    def wb_t(k, _):
        read_acc_scaled(k, 1)
        pltpu.sync_copy(bufa, tth.at[pl.ds(toff + base + k * _CH, _CH)])
        return 0
    lax.fori_loop(0, _KCH, wb_t, 0)
    plsc.subcore_barrier()
    stage_idx(sn_h, dn_h)
    scatter_pass(t2)
    plsc.subcore_barrier()
    def wb_neg(k, _):
        read_acc_scaled(k, 3)
        add_from(tth, k)
        add_from(n1h, k)
        add_from(e_neg, k, coloff=coff)
        scale_alpha()
        pltpu.sync_copy(
            bufa, neg_out.at[pl.ds(base + k * _CH, _CH), pl.ds(coff, _HALF)])
        return 0
    lax.fori_loop(0, _KCH, wb_neg, 0)


@functools.cache
def _build():
    mesh = plsc.VectorSubcoreMesh(core_axis_name="c", subcore_axis_name="s")
    tbl = pltpu.HBM((_NC * _NPAD, _HALF), jnp.float32)
    return functools.partial(
        pl.kernel,
        mesh=mesh,
        out_type=(jax.ShapeDtypeStruct((_NPAD, _DIM), jnp.float32),
                  jax.ShapeDtypeStruct((_NPAD, _DIM), jnp.float32)),
        scratch_types=[
            tbl, tbl, tbl, tbl, tbl, tbl, tbl,
            pltpu.HBM((_NC * 4 * _NPAD * 16,), jnp.float32),
            pltpu.VMEM((_EPT,), jnp.int32),
            pltpu.VMEM((_EPT,), jnp.int32),
            pltpu.VMEM((_CH, _HALF), jnp.float32),
            pltpu.VMEM((_CH, _HALF), jnp.float32),
            pltpu.VMEM((_SCW,), jnp.float32),
            pltpu.VMEM((_SCW,), jnp.float32),
            pltpu.VMEM((_SCW,), jnp.float32),
            pltpu.VMEM((_SCW,), jnp.float32),
            pltpu.VMEM((8, _HALF), jnp.float32),
            pltpu.VMEM((16, _HALF), jnp.float32),
            pltpu.VMEM_SHARED((_NPAD, _HALF), jnp.float32),
            pltpu.SemaphoreType.DMA((2,)),
            pltpu.SemaphoreType.DMA((2,)),
        ],
    )(_body)


def _prep_idx(row):
    r = row.astype(jnp.int32).reshape(_NS, _EPW)
    r = jnp.pad(r, ((0, 0), (0, _EPT - _EPW)), constant_values=_NN)
    return r.reshape(_NS * _EPT)


def kernel(user_embedding, item_embedding, user_neg_embedding,
           item_neg_embedding, edge_index_p, edge_index_n):
    e_pos = jnp.concatenate([user_embedding, item_embedding], axis=0)
    e_neg = jnp.concatenate([user_neg_embedding, item_neg_embedding], axis=0)
    e_pos = jnp.pad(e_pos, ((0, _NPAD - _NN), (0, 0)))
    e_neg = jnp.pad(e_neg, ((0, _NPAD - _NN), (0, 0)))
    sp, dp = _prep_idx(edge_index_p[0]), _prep_idx(edge_index_p[1])
    sn, dn = _prep_idx(edge_index_n[0]), _prep_idx(edge_index_n[1])
    pos, neg = _build()(e_pos, e_neg, sp, dp, sn, dn)
    return pos[:_NN], neg[:_NN]


# speedup vs baseline: 3.9675x; 1.0538x over previous
import functools

import jax
import jax.numpy as jnp
from jax import lax
from jax.experimental import pallas as pl
from jax.experimental.pallas import tpu as pltpu
from jax.experimental.pallas import tpu_sc as plsc

_M, _NV, _DIM = 2000, 8000, 256
_NN = _M + _NV
_E = 160000
_NC, _NS = 2, 16
_NPAD = 10240
_RPT = _NPAD // _NS
_CH = 64
_KCH = _RPT // _CH
_EPW = _E // _NS
_EPT = 10240
_ECH = _EPT // _CH
_HALF = 128
_NCOL = _HALF // 16
_SUB = _CH // 16
_SCW = _CH * 16
_ALPHA = 1.0 / 3.0


def _rsqrt_newton(x):
    y = jnp.full((16,), 0.70710678 * 2.0 ** -8, jnp.float32)
    for k in range(7, -1, -1):
        thr = jnp.full((16,), 4.0 ** (k + 1), jnp.float32)
        y = jnp.where(x < thr, jnp.full((16,), 0.70710678 * 2.0 ** -k, jnp.float32), y)
    c15 = jnp.full((16,), 1.5, jnp.float32)
    ch = jnp.full((16,), 0.5, jnp.float32)
    for _ in range(5):
        y = y * (c15 - ch * x * y * y)
    return y


def _body(e_pos, e_neg, sp_h, dp_h, sn_h, dn_h,
          pos_out, neg_out,
          t0, t1, t2, t3, p1h, n1h, tth, sc_h,
          svv, dvv,
          bufa, bufb,
          sb0, sb1, sb2, sb3,
          zbuf, onesb,
          acc,
          semg, sems):
    cid = lax.axis_index("c")
    sid = lax.axis_index("s")
    base = sid * _RPT
    coff = cid * _HALF
    toff = cid * _NPAD
    scb = (cid * 4) * _NPAD * 16

    def stage_idx(src_hbm, dst_hbm):
        pltpu.sync_copy(src_hbm.at[pl.ds(sid * _EPT, _EPT)], svv)
        pltpu.sync_copy(dst_hbm.at[pl.ds(sid * _EPT, _EPT)], dvv)

    def zfill(r, _):
        zv = jnp.zeros((16,), jnp.float32)
        for cc in range(_NCOL):
            zbuf[r, pl.ds(cc * 16, 16)] = zv
        return 0
    lax.fori_loop(0, 8, zfill, 0)

    def zero_acc_chunk(k):
        for q in range(_CH // 8):
            pltpu.sync_copy(zbuf, acc.at[pl.ds(base + k * _CH + q * 8, 8)])

    def zinit(k, _):
        zero_acc_chunk(k)
        return 0
    lax.fori_loop(0, _KCH, zinit, 0)
    plsc.subcore_barrier()

    for rnd, (s_hbm, d_hbm) in ((0, (sp_h, dp_h)), (1, (sn_h, dn_h))):
        stage_idx(s_hbm, d_hbm)
        for half, iv in ((0, svv), (1, dvv)):
            side = rnd * 2 + half
            def bandfill(r, _, side=side):
                for cc in range(_NCOL):
                    v = 1.0 if cc == side else 0.0
                    onesb[r, pl.ds(cc * 16, 16)] = jnp.full((16,), v, jnp.float32)
                return 0
            lax.fori_loop(0, 16, bandfill, 0)
            def ddrain(slot):
                for j in range(_SUB):
                    pltpu.make_async_copy(
                        onesb.at[pl.ds(0, 16)],
                        acc.at[jnp.zeros((16,), jnp.int32)],
                        semg.at[slot]).wait()
            def dquad(q, _, iv=iv):
                g4 = q * 4
                for par in range(4):
                    g = g4 + par
                    @pl.when(g >= 4)
                    def _():
                        ddrain(par)
                    for j in range(_SUB):
                        dv = iv[pl.ds(g * _CH + j * 16, 16)]
                        pltpu.async_copy(onesb.at[pl.ds(0, 16)], acc.at[dv],
                                         semg.at[par], add=True)
                return 0
            lax.fori_loop(0, _ECH // 4, dquad, 0)
            for par in range(4):
                ddrain(par)
    plsc.subcore_barrier()

    def scprod(k, _):
        pltpu.sync_copy(acc.at[pl.ds(base + k * _CH, _CH)], bufa)
        zero_acc_chunk(k)
        def abstep(r, _):
            for side, sref in ((0, sb0), (1, sb1), (2, sb2), (3, sb3)):
                d = jnp.maximum(bufa[r, pl.ds(side * 16, 16)],
                                jnp.full((16,), 1.0, jnp.float32))
                sref[pl.ds(r * 16, 16)] = _rsqrt_newton(d)
            return 0
        lax.fori_loop(0, _CH, abstep, 0)
        for side, sref in ((0, sb0), (1, sb1), (2, sb2), (3, sb3)):
            pltpu.sync_copy(
                sref,
                sc_h.at[pl.ds(scb + side * _NPAD * 16 + (base + k * _CH) * 16,
                              _SCW)])
        return 0
    lax.fori_loop(0, _KCH, scprod, 0)
    plsc.subcore_barrier()

    def fetch_scale(sref, side, k):
        pltpu.sync_copy(
            sc_h.at[pl.ds(scb + side * _NPAD * 16 + (base + k * _CH) * 16,
                          _SCW)], sref)

    def scale_to(dst_ref, src_ref, sref):
        def grp(r, _):
            sv = sref[pl.ds(r * 16, 16)]
            for cc in range(_NCOL):
                dst_ref[r, pl.ds(cc * 16, 16)] = src_ref[r, pl.ds(cc * 16, 16)] * sv
            return 0
        lax.fori_loop(0, _CH, grp, 0)

    def prescale_two(src_h, dsta_h, sidea, dstb_h, sideb):
        def pk(k, _):
            pltpu.sync_copy(
                src_h.at[pl.ds(base + k * _CH, _CH), pl.ds(coff, _HALF)], bufa)
            fetch_scale(sb0, sidea, k)
            fetch_scale(sb1, sideb, k)
            scale_to(bufb, bufa, sb0)
            pltpu.sync_copy(bufb, dsta_h.at[pl.ds(toff + base + k * _CH, _CH)])
            scale_to(bufb, bufa, sb1)
            pltpu.sync_copy(bufb, dstb_h.at[pl.ds(toff + base + k * _CH, _CH)])
            return 0
        lax.fori_loop(0, _KCH, pk, 0)

    def scatter_pass(tbl_h):
        C2 = 32
        E2 = _EPT // C2
        S2 = C2 // 16

        def rq(slot, j):
            b = bufa if slot < 2 else bufb
            return b.at[pl.ds((slot % 2) * C2 + j * 16, 16)]

        def issue_gather(g, slot):
            for j in range(S2):
                iv = svv[pl.ds(g * C2 + j * 16, 16)] + jnp.full(
                    (16,), 1, jnp.int32) * toff
                pltpu.async_copy(tbl_h.at[iv], rq(slot, j), semg.at[slot])

        def wait_gather(slot):
            for j in range(S2):
                pltpu.make_async_copy(tbl_h.at[jnp.zeros((16,), jnp.int32)],
                                      rq(slot, j), semg.at[slot]).wait()

        def issue_scatter(g, slot):
            for j in range(S2):
                dv = dvv[pl.ds(g * C2 + j * 16, 16)]
                pltpu.async_copy(rq(slot, j), acc.at[dv], sems.at[slot],
                                 add=True)

        def drain_scatter(slot):
            for j in range(S2):
                pltpu.make_async_copy(rq(slot, j),
                                      acc.at[jnp.zeros((16,), jnp.int32)],
                                      sems.at[slot]).wait()

        issue_gather(0, 0)
        issue_gather(1, 1)
        def quad(q, _):
            g4 = q * 4
            for par in range(4):
                g = g4 + par
                nslot = (par + 2) % 4
                wait_gather(par)
                issue_scatter(g, par)
                @pl.when(g + 2 < E2)
                def _():
                    @pl.when(g >= 2)
                    def _():
                        drain_scatter(nslot)
                    issue_gather(g + 2, nslot)
            return 0
        lax.fori_loop(0, E2 // 4, quad, 0)
        for par in range(4):
            drain_scatter(par)

    def read_acc_scaled(k, side):
        pltpu.sync_copy(acc.at[pl.ds(base + k * _CH, _CH)], bufa)
        zero_acc_chunk(k)
        fetch_scale(sb0, side, k)
        scale_to(bufa, bufa, sb0)

    def add_from(src_h, k, coloff=None):
        if coloff is None:
            pltpu.sync_copy(src_h.at[pl.ds(toff + base + k * _CH, _CH)], bufb)
        else:
            pltpu.sync_copy(
                src_h.at[pl.ds(base + k * _CH, _CH), pl.ds(coloff, _HALF)], bufb)
        def grp(r, _):
            for cc in range(_NCOL):
                bufa[r, pl.ds(cc * 16, 16)] = (
                    bufa[r, pl.ds(cc * 16, 16)] + bufb[r, pl.ds(cc * 16, 16)])
            return 0
        lax.fori_loop(0, _CH, grp, 0)

    def scale_alpha():
        av = jnp.full((16,), _ALPHA, jnp.float32)
        def grp(r, _):
            for cc in range(_NCOL):
                bufa[r, pl.ds(cc * 16, 16)] = bufa[r, pl.ds(cc * 16, 16)] * av
            return 0
        lax.fori_loop(0, _CH, grp, 0)

    prescale_two(e_pos, t0, 0, t1, 2)
    plsc.subcore_barrier()

    stage_idx(sp_h, dp_h)
    scatter_pass(t0)
    plsc.subcore_barrier()
    def wb_l0p(k, _):
        read_acc_scaled(k, 1)
        pltpu.sync_copy(bufa, p1h.at[pl.ds(toff + base + k * _CH, _CH)])
        fetch_scale(sb1, 0, k)
        scale_to(bufb, bufa, sb1)
        pltpu.sync_copy(bufb, t0.at[pl.ds(toff + base + k * _CH, _CH)])
        fetch_scale(sb1, 2, k)
        scale_to(bufb, bufa, sb1)
        pltpu.sync_copy(bufb, t2.at[pl.ds(toff + base + k * _CH, _CH)])
        return 0
    lax.fori_loop(0, _KCH, wb_l0p, 0)
    plsc.subcore_barrier()

    stage_idx(sn_h, dn_h)
    scatter_pass(t1)
    plsc.subcore_barrier()
    def wb_l0n(k, _):
        read_acc_scaled(k, 3)
        pltpu.sync_copy(bufa, n1h.at[pl.ds(toff + base + k * _CH, _CH)])
        fetch_scale(sb1, 2, k)
        scale_to(bufb, bufa, sb1)
        pltpu.sync_copy(bufb, t1.at[pl.ds(toff + base + k * _CH, _CH)])
        fetch_scale(sb1, 0, k)
        scale_to(bufb, bufa, sb1)
        pltpu.sync_copy(bufb, t3.at[pl.ds(toff + base + k * _CH, _CH)])
        return 0
    lax.fori_loop(0, _KCH, wb_l0n, 0)
    plsc.subcore_barrier()

    stage_idx(sp_h, dp_h)
    scatter_pass(t0)
    plsc.subcore_barrier()
    def wb_t(k, _):
        read_acc_scaled(k, 1)
        pltpu.sync_copy(bufa, tth.at[pl.ds(toff + base + k * _CH, _CH)])
        return 0
    lax.fori_loop(0, _KCH, wb_t, 0)
    plsc.subcore_barrier()
    stage_idx(sn_h, dn_h)
    scatter_pass(t1)
    plsc.subcore_barrier()
    def wb_pos(k, _):
        read_acc_scaled(k, 3)
        add_from(tth, k)
        add_from(p1h, k)
        add_from(e_pos, k, coloff=coff)
        scale_alpha()
        pltpu.sync_copy(
            bufa, pos_out.at[pl.ds(base + k * _CH, _CH), pl.ds(coff, _HALF)])
        return 0
    lax.fori_loop(0, _KCH, wb_pos, 0)
    plsc.subcore_barrier()

    stage_idx(sp_h, dp_h)
    scatter_pass(t3)
    plsc.subcore_barrier()
    def wb_t(k, _):
        read_acc_scaled(k, 1)
        pltpu.sync_copy(bufa, tth.at[pl.ds(toff + base + k * _CH, _CH)])
        return 0
    lax.fori_loop(0, _KCH, wb_t, 0)
    plsc.subcore_barrier()
    stage_idx(sn_h, dn_h)
    scatter_pass(t2)
    plsc.subcore_barrier()
    def wb_neg(k, _):
        read_acc_scaled(k, 3)
        add_from(tth, k)
        add_from(n1h, k)
        add_from(e_neg, k, coloff=coff)
        scale_alpha()
        pltpu.sync_copy(
            bufa, neg_out.at[pl.ds(base + k * _CH, _CH), pl.ds(coff, _HALF)])
        return 0
    lax.fori_loop(0, _KCH, wb_neg, 0)


@functools.cache
def _build():
    mesh = plsc.VectorSubcoreMesh(core_axis_name="c", subcore_axis_name="s")
    tbl = pltpu.HBM((_NC * _NPAD, _HALF), jnp.float32)
    return functools.partial(
        pl.kernel,
        mesh=mesh,
        out_type=(jax.ShapeDtypeStruct((_NPAD, _DIM), jnp.float32),
                  jax.ShapeDtypeStruct((_NPAD, _DIM), jnp.float32)),
        scratch_types=[
            tbl, tbl, tbl, tbl, tbl, tbl, tbl,
            pltpu.HBM((_NC * 4 * _NPAD * 16,), jnp.float32),
            pltpu.VMEM((_EPT,), jnp.int32),
            pltpu.VMEM((_EPT,), jnp.int32),
            pltpu.VMEM((_CH, _HALF), jnp.float32),
            pltpu.VMEM((_CH, _HALF), jnp.float32),
            pltpu.VMEM((_SCW,), jnp.float32),
            pltpu.VMEM((_SCW,), jnp.float32),
            pltpu.VMEM((_SCW,), jnp.float32),
            pltpu.VMEM((_SCW,), jnp.float32),
            pltpu.VMEM((8, _HALF), jnp.float32),
            pltpu.VMEM((16, _HALF), jnp.float32),
            pltpu.VMEM_SHARED((_NPAD, _HALF), jnp.float32),
            pltpu.SemaphoreType.DMA((4,)),
            pltpu.SemaphoreType.DMA((4,)),
        ],
    )(_body)


def _prep_idx(row):
    r = row.astype(jnp.int32).reshape(_NS, _EPW)
    r = jnp.pad(r, ((0, 0), (0, _EPT - _EPW)), constant_values=_NN)
    return r.reshape(_NS * _EPT)


def kernel(user_embedding, item_embedding, user_neg_embedding,
           item_neg_embedding, edge_index_p, edge_index_n):
    e_pos = jnp.concatenate([user_embedding, item_embedding], axis=0)
    e_neg = jnp.concatenate([user_neg_embedding, item_neg_embedding], axis=0)
    e_pos = jnp.pad(e_pos, ((0, _NPAD - _NN), (0, 0)))
    e_neg = jnp.pad(e_neg, ((0, _NPAD - _NN), (0, 0)))
    sp, dp = _prep_idx(edge_index_p[0]), _prep_idx(edge_index_p[1])
    sn, dn = _prep_idx(edge_index_n[0]), _prep_idx(edge_index_n[1])
    pos, neg = _build()(e_pos, e_neg, sp, dp, sn, dn)
    return pos[:_NN], neg[:_NN]


# speedup vs baseline: 4.0631x; 1.0241x over previous
import functools

import jax
import jax.numpy as jnp
from jax import lax
from jax.experimental import pallas as pl
from jax.experimental.pallas import tpu as pltpu
from jax.experimental.pallas import tpu_sc as plsc

_M, _NV, _DIM = 2000, 8000, 256
_NN = _M + _NV
_E = 160000
_NC, _NS = 2, 16
_NPAD = 10240
_RPT = _NPAD // _NS
_CH = 64
_KCH = _RPT // _CH
_EPW = _E // _NS
_EPT = 10240
_ECH = _EPT // _CH
_HALF = 128
_NCOL = _HALF // 16
_SUB = _CH // 16
_SCW = _CH * 16
_ALPHA = 1.0 / 3.0


def _rsqrt_newton(x):
    y = jnp.full((16,), 0.70710678 * 2.0 ** -8, jnp.float32)
    for k in range(7, -1, -1):
        thr = jnp.full((16,), 4.0 ** (k + 1), jnp.float32)
        y = jnp.where(x < thr, jnp.full((16,), 0.70710678 * 2.0 ** -k, jnp.float32), y)
    c15 = jnp.full((16,), 1.5, jnp.float32)
    ch = jnp.full((16,), 0.5, jnp.float32)
    for _ in range(5):
        y = y * (c15 - ch * x * y * y)
    return y


def _body(e_pos, e_neg, sp_h, dp_h, sn_h, dn_h,
          pos_out, neg_out,
          t0, t1, t2, t3, p1h, n1h, tth, sc_h,
          svv, dvv,
          bufa, bufb,
          sb0, sb1, sb2, sb3,
          zbuf, onesb,
          acc,
          semg, sems, semz):
    cid = lax.axis_index("c")
    sid = lax.axis_index("s")
    base = sid * _RPT
    coff = cid * _HALF
    toff = cid * _NPAD
    scb = (cid * 4) * _NPAD * 16

    def stage_idx(src_hbm, dst_hbm):
        pltpu.sync_copy(src_hbm.at[pl.ds(sid * _EPT, _EPT)], svv)
        pltpu.sync_copy(dst_hbm.at[pl.ds(sid * _EPT, _EPT)], dvv)

    def zfill(r, _):
        zv = jnp.zeros((16,), jnp.float32)
        for cc in range(_NCOL):
            zbuf[r, pl.ds(cc * 16, 16)] = zv
        return 0
    lax.fori_loop(0, 8, zfill, 0)

    def zero_acc_chunk(k):
        for q in range(_CH // 8):
            pltpu.async_copy(zbuf, acc.at[pl.ds(base + k * _CH + q * 8, 8)],
                             semz)

    def zdrain(nchunks):
        def zd(i, _):
            for q in range(_CH // 8):
                pltpu.make_async_copy(zbuf, acc.at[pl.ds(base, 8)], semz).wait()
            return 0
        lax.fori_loop(0, nchunks, zd, 0)

    def zinit(k, _):
        zero_acc_chunk(k)
        return 0
    lax.fori_loop(0, _KCH, zinit, 0)
    zdrain(_KCH)
    plsc.subcore_barrier()

    for rnd, (s_hbm, d_hbm) in ((0, (sp_h, dp_h)), (1, (sn_h, dn_h))):
        stage_idx(s_hbm, d_hbm)
        for half, iv in ((0, svv), (1, dvv)):
            side = rnd * 2 + half
            def bandfill(r, _, side=side):
                for cc in range(_NCOL):
                    v = 1.0 if cc == side else 0.0
                    onesb[r, pl.ds(cc * 16, 16)] = jnp.full((16,), v, jnp.float32)
                return 0
            lax.fori_loop(0, 16, bandfill, 0)
            def ddrain(slot):
                for j in range(_SUB):
                    pltpu.make_async_copy(
                        onesb.at[pl.ds(0, 16)],
                        acc.at[jnp.zeros((16,), jnp.int32)],
                        semg.at[slot]).wait()
            def dquad(q, _, iv=iv):
                g4 = q * 4
                for par in range(4):
                    g = g4 + par
                    @pl.when(g >= 4)
                    def _():
                        ddrain(par)
                    for j in range(_SUB):
                        dv = iv[pl.ds(g * _CH + j * 16, 16)]
                        pltpu.async_copy(onesb.at[pl.ds(0, 16)], acc.at[dv],
                                         semg.at[par], add=True)
                return 0
            lax.fori_loop(0, _ECH // 4, dquad, 0)
            for par in range(4):
                ddrain(par)
    plsc.subcore_barrier()

    def scprod(k, _):
        pltpu.sync_copy(acc.at[pl.ds(base + k * _CH, _CH)], bufa)
        zero_acc_chunk(k)
        def abstep(r, _):
            for side, sref in ((0, sb0), (1, sb1), (2, sb2), (3, sb3)):
                d = jnp.maximum(bufa[r, pl.ds(side * 16, 16)],
                                jnp.full((16,), 1.0, jnp.float32))
                sref[pl.ds(r * 16, 16)] = _rsqrt_newton(d)
            return 0
        lax.fori_loop(0, _CH, abstep, 0)
        for side, sref in ((0, sb0), (1, sb1), (2, sb2), (3, sb3)):
            pltpu.sync_copy(
                sref,
                sc_h.at[pl.ds(scb + side * _NPAD * 16 + (base + k * _CH) * 16,
                              _SCW)])
        return 0
    lax.fori_loop(0, _KCH, scprod, 0)
    zdrain(_KCH)
    plsc.subcore_barrier()

    def fetch_scale(sref, side, k):
        pltpu.sync_copy(
            sc_h.at[pl.ds(scb + side * _NPAD * 16 + (base + k * _CH) * 16,
                          _SCW)], sref)

    def scale_to(dst_ref, src_ref, sref):
        def grp(r, _):
            sv = sref[pl.ds(r * 16, 16)]
            for cc in range(_NCOL):
                dst_ref[r, pl.ds(cc * 16, 16)] = src_ref[r, pl.ds(cc * 16, 16)] * sv
            return 0
        lax.fori_loop(0, _CH, grp, 0)

    def prescale_two(src_h, dsta_h, sidea, dstb_h, sideb):
        def pk(k, _):
            pltpu.sync_copy(
                src_h.at[pl.ds(base + k * _CH, _CH), pl.ds(coff, _HALF)], bufa)
            fetch_scale(sb0, sidea, k)
            fetch_scale(sb1, sideb, k)
            scale_to(bufb, bufa, sb0)
            pltpu.sync_copy(bufb, dsta_h.at[pl.ds(toff + base + k * _CH, _CH)])
            scale_to(bufb, bufa, sb1)
            pltpu.sync_copy(bufb, dstb_h.at[pl.ds(toff + base + k * _CH, _CH)])
            return 0
        lax.fori_loop(0, _KCH, pk, 0)

    def scatter_pass(tbl_h):
        C2 = 32
        E2 = _EPT // C2
        S2 = C2 // 16

        def rq(slot, j):
            b = bufa if slot < 2 else bufb
            return b.at[pl.ds((slot % 2) * C2 + j * 16, 16)]

        def issue_gather(g, slot):
            for j in range(S2):
                iv = svv[pl.ds(g * C2 + j * 16, 16)] + jnp.full(
                    (16,), 1, jnp.int32) * toff
                pltpu.async_copy(tbl_h.at[iv], rq(slot, j), semg.at[slot])

        def wait_gather(slot):
            for j in range(S2):
                pltpu.make_async_copy(tbl_h.at[jnp.zeros((16,), jnp.int32)],
                                      rq(slot, j), semg.at[slot]).wait()

        def issue_scatter(g, slot):
            for j in range(S2):
                dv = dvv[pl.ds(g * C2 + j * 16, 16)]
                pltpu.async_copy(rq(slot, j), acc.at[dv], sems.at[slot],
                                 add=True)

        def drain_scatter(slot):
            for j in range(S2):
                pltpu.make_async_copy(rq(slot, j),
                                      acc.at[jnp.zeros((16,), jnp.int32)],
                                      sems.at[slot]).wait()

        issue_gather(0, 0)
        issue_gather(1, 1)
        def quad(q, _):
            g4 = q * 4
            for par in range(4):
                g = g4 + par
                nslot = (par + 2) % 4
                wait_gather(par)
                issue_scatter(g, par)
                @pl.when(g + 2 < E2)
                def _():
                    @pl.when(g >= 2)
                    def _():
                        drain_scatter(nslot)
                    issue_gather(g + 2, nslot)
            return 0
        lax.fori_loop(0, E2 // 4, quad, 0)
        for par in range(4):
            drain_scatter(par)

    def read_acc_scaled(k, side):
        pltpu.sync_copy(acc.at[pl.ds(base + k * _CH, _CH)], bufa)
        zero_acc_chunk(k)
        fetch_scale(sb0, side, k)
        scale_to(bufa, bufa, sb0)

    def add_from(src_h, k, coloff=None):
        if coloff is None:
            pltpu.sync_copy(src_h.at[pl.ds(toff + base + k * _CH, _CH)], bufb)
        else:
            pltpu.sync_copy(
                src_h.at[pl.ds(base + k * _CH, _CH), pl.ds(coloff, _HALF)], bufb)
        def grp(r, _):
            for cc in range(_NCOL):
                bufa[r, pl.ds(cc * 16, 16)] = (
                    bufa[r, pl.ds(cc * 16, 16)] + bufb[r, pl.ds(cc * 16, 16)])
            return 0
        lax.fori_loop(0, _CH, grp, 0)

    def scale_alpha():
        av = jnp.full((16,), _ALPHA, jnp.float32)
        def grp(r, _):
            for cc in range(_NCOL):
                bufa[r, pl.ds(cc * 16, 16)] = bufa[r, pl.ds(cc * 16, 16)] * av
            return 0
        lax.fori_loop(0, _CH, grp, 0)

    prescale_two(e_pos, t0, 0, t1, 2)
    plsc.subcore_barrier()

    stage_idx(sp_h, dp_h)
    scatter_pass(t0)
    plsc.subcore_barrier()
    def wb_l0p(k, _):
        read_acc_scaled(k, 1)
        pltpu.sync_copy(bufa, p1h.at[pl.ds(toff + base + k * _CH, _CH)])
        fetch_scale(sb1, 0, k)
        scale_to(bufb, bufa, sb1)
        pltpu.sync_copy(bufb, t0.at[pl.ds(toff + base + k * _CH, _CH)])
        fetch_scale(sb1, 2, k)
        scale_to(bufb, bufa, sb1)
        pltpu.sync_copy(bufb, t2.at[pl.ds(toff + base + k * _CH, _CH)])
        return 0
    lax.fori_loop(0, _KCH, wb_l0p, 0)
    zdrain(_KCH)
    plsc.subcore_barrier()

    stage_idx(sn_h, dn_h)
    scatter_pass(t1)
    plsc.subcore_barrier()
    def wb_l0n(k, _):
        read_acc_scaled(k, 3)
        pltpu.sync_copy(bufa, n1h.at[pl.ds(toff + base + k * _CH, _CH)])
        fetch_scale(sb1, 2, k)
        scale_to(bufb, bufa, sb1)
        pltpu.sync_copy(bufb, t1.at[pl.ds(toff + base + k * _CH, _CH)])
        fetch_scale(sb1, 0, k)
        scale_to(bufb, bufa, sb1)
        pltpu.sync_copy(bufb, t3.at[pl.ds(toff + base + k * _CH, _CH)])
        return 0
    lax.fori_loop(0, _KCH, wb_l0n, 0)
    zdrain(_KCH)
    plsc.subcore_barrier()

    stage_idx(sp_h, dp_h)
    scatter_pass(t0)
    plsc.subcore_barrier()
    def wb_t(k, _):
        read_acc_scaled(k, 1)
        pltpu.sync_copy(bufa, tth.at[pl.ds(toff + base + k * _CH, _CH)])
        return 0
    lax.fori_loop(0, _KCH, wb_t, 0)
    zdrain(_KCH)
    plsc.subcore_barrier()
    stage_idx(sn_h, dn_h)
    scatter_pass(t1)
    plsc.subcore_barrier()
    def wb_pos(k, _):
        read_acc_scaled(k, 3)
        add_from(tth, k)
        add_from(p1h, k)
        add_from(e_pos, k, coloff=coff)
        scale_alpha()
        pltpu.sync_copy(
            bufa, pos_out.at[pl.ds(base + k * _CH, _CH), pl.ds(coff, _HALF)])
        return 0
    lax.fori_loop(0, _KCH, wb_pos, 0)
    zdrain(_KCH)
    plsc.subcore_barrier()

    stage_idx(sp_h, dp_h)
    scatter_pass(t3)
    plsc.subcore_barrier()
    def wb_t(k, _):
        read_acc_scaled(k, 1)
        pltpu.sync_copy(bufa, tth.at[pl.ds(toff + base + k * _CH, _CH)])
        return 0
    lax.fori_loop(0, _KCH, wb_t, 0)
    zdrain(_KCH)
    plsc.subcore_barrier()
    stage_idx(sn_h, dn_h)
    scatter_pass(t2)
    plsc.subcore_barrier()
    def wb_neg(k, _):
        read_acc_scaled(k, 3)
        add_from(tth, k)
        add_from(n1h, k)
        add_from(e_neg, k, coloff=coff)
        scale_alpha()
        pltpu.sync_copy(
            bufa, neg_out.at[pl.ds(base + k * _CH, _CH), pl.ds(coff, _HALF)])
        return 0
    lax.fori_loop(0, _KCH, wb_neg, 0)
    zdrain(_KCH)


@functools.cache
def _build():
    mesh = plsc.VectorSubcoreMesh(core_axis_name="c", subcore_axis_name="s")
    tbl = pltpu.HBM((_NC * _NPAD, _HALF), jnp.float32)
    return functools.partial(
        pl.kernel,
        mesh=mesh,
        out_type=(jax.ShapeDtypeStruct((_NPAD, _DIM), jnp.float32),
                  jax.ShapeDtypeStruct((_NPAD, _DIM), jnp.float32)),
        scratch_types=[
            tbl, tbl, tbl, tbl, tbl, tbl, tbl,
            pltpu.HBM((_NC * 4 * _NPAD * 16,), jnp.float32),
            pltpu.VMEM((_EPT,), jnp.int32),
            pltpu.VMEM((_EPT,), jnp.int32),
            pltpu.VMEM((_CH, _HALF), jnp.float32),
            pltpu.VMEM((_CH, _HALF), jnp.float32),
            pltpu.VMEM((_SCW,), jnp.float32),
            pltpu.VMEM((_SCW,), jnp.float32),
            pltpu.VMEM((_SCW,), jnp.float32),
            pltpu.VMEM((_SCW,), jnp.float32),
            pltpu.VMEM((8, _HALF), jnp.float32),
            pltpu.VMEM((16, _HALF), jnp.float32),
            pltpu.VMEM_SHARED((_NPAD, _HALF), jnp.float32),
            pltpu.SemaphoreType.DMA((4,)),
            pltpu.SemaphoreType.DMA((4,)),
            pltpu.SemaphoreType.DMA,
        ],
    )(_body)


def _prep_idx(row):
    r = row.astype(jnp.int32).reshape(_NS, _EPW)
    r = jnp.pad(r, ((0, 0), (0, _EPT - _EPW)), constant_values=_NN)
    return r.reshape(_NS * _EPT)


def kernel(user_embedding, item_embedding, user_neg_embedding,
           item_neg_embedding, edge_index_p, edge_index_n):
    e_pos = jnp.concatenate([user_embedding, item_embedding], axis=0)
    e_neg = jnp.concatenate([user_neg_embedding, item_neg_embedding], axis=0)
    e_pos = jnp.pad(e_pos, ((0, _NPAD - _NN), (0, 0)))
    e_neg = jnp.pad(e_neg, ((0, _NPAD - _NN), (0, 0)))
    sp, dp = _prep_idx(edge_index_p[0]), _prep_idx(edge_index_p[1])
    sn, dn = _prep_idx(edge_index_n[0]), _prep_idx(edge_index_n[1])
    pos, neg = _build()(e_pos, e_neg, sp, dp, sn, dn)
    return pos[:_NN], neg[:_NN]


# speedup vs baseline: 4.2027x; 1.0343x over previous
import functools

import jax
import jax.numpy as jnp
from jax import lax
from jax.experimental import pallas as pl
from jax.experimental.pallas import tpu as pltpu
from jax.experimental.pallas import tpu_sc as plsc

_M, _NV, _DIM = 2000, 8000, 256
_NN = _M + _NV
_E = 160000
_NC, _NS = 2, 16
_NPAD = 10240
_RPT = _NPAD // _NS
_CH = 64
_KCH = _RPT // _CH
_EPW = _E // _NS
_EPT = 10240
_ECH = _EPT // _CH
_HALF = 128
_NCOL = _HALF // 16
_SUB = _CH // 16
_SCW = _CH * 16
_ALPHA = 1.0 / 3.0


def _rsqrt_newton(x):
    y = jnp.full((16,), 0.70710678 * 2.0 ** -8, jnp.float32)
    for k in range(7, -1, -1):
        thr = jnp.full((16,), 4.0 ** (k + 1), jnp.float32)
        y = jnp.where(x < thr, jnp.full((16,), 0.70710678 * 2.0 ** -k, jnp.float32), y)
    c15 = jnp.full((16,), 1.5, jnp.float32)
    ch = jnp.full((16,), 0.5, jnp.float32)
    for _ in range(5):
        y = y * (c15 - ch * x * y * y)
    return y


def _body(e_pos, e_neg, sp_h, dp_h, sn_h, dn_h,
          pos_out, neg_out,
          t0, t1, t2, t3, p1h, n1h, tth, sc_h,
          svv, dvv,
          bufa, bufb,
          sb0, sb1, sb2, sb3,
          zbuf, onesb,
          acc,
          semg, sems, semz):
    cid = lax.axis_index("c")
    sid = lax.axis_index("s")
    base = sid * _RPT
    coff = cid * _HALF
    toff = cid * _NPAD
    scb = (cid * 4) * _NPAD * 16

    def stage_idx(src_hbm, dst_hbm):
        pltpu.sync_copy(src_hbm.at[pl.ds(sid * _EPT, _EPT)], svv)
        pltpu.sync_copy(dst_hbm.at[pl.ds(sid * _EPT, _EPT)], dvv)

    def zfill(r, _):
        zv = jnp.zeros((16,), jnp.float32)
        for cc in range(_NCOL):
            zbuf[r, pl.ds(cc * 16, 16)] = zv
        return 0
    lax.fori_loop(0, 8, zfill, 0)

    def zero_acc_chunk(k):
        for q in range(_CH // 8):
            pltpu.async_copy(zbuf, acc.at[pl.ds(base + k * _CH + q * 8, 8)],
                             semz)

    def zdrain(nchunks):
        def zd(i, _):
            for q in range(_CH // 8):
                pltpu.make_async_copy(zbuf, acc.at[pl.ds(base, 8)], semz).wait()
            return 0
        lax.fori_loop(0, nchunks, zd, 0)

    def zinit(k, _):
        zero_acc_chunk(k)
        return 0
    lax.fori_loop(0, _KCH, zinit, 0)
    zdrain(_KCH)
    plsc.subcore_barrier()

    for rnd, (s_hbm, d_hbm) in ((0, (sp_h, dp_h)), (1, (sn_h, dn_h))):
        stage_idx(s_hbm, d_hbm)
        for half, iv in ((0, svv), (1, dvv)):
            side = rnd * 2 + half
            def bandfill(r, _, side=side):
                for cc in range(_NCOL):
                    v = 1.0 if cc == side else 0.0
                    onesb[r, pl.ds(cc * 16, 16)] = jnp.full((16,), v, jnp.float32)
                return 0
            lax.fori_loop(0, 16, bandfill, 0)
            def ddrain(slot):
                for j in range(_SUB):
                    pltpu.make_async_copy(
                        onesb.at[pl.ds(0, 16)],
                        acc.at[jnp.zeros((16,), jnp.int32)],
                        semg.at[slot]).wait()
            def dquad(q, _, iv=iv):
                g4 = q * 4
                for par in range(4):
                    g = g4 + par
                    @pl.when(g >= 4)
                    def _():
                        ddrain(par)
                    for j in range(_SUB):
                        dv = iv[pl.ds(g * _CH + j * 16, 16)]
                        pltpu.async_copy(onesb.at[pl.ds(0, 16)], acc.at[dv],
                                         semg.at[par], add=True)
                return 0
            lax.fori_loop(0, _ECH // 4, dquad, 0)
            for par in range(4):
                ddrain(par)
    plsc.subcore_barrier()

    def scprod(k, _):
        pltpu.sync_copy(acc.at[pl.ds(base + k * _CH, _CH)], bufa)
        zero_acc_chunk(k)
        def abstep(r, _):
            for side, sref in ((0, sb0), (1, sb1), (2, sb2), (3, sb3)):
                d = jnp.maximum(bufa[r, pl.ds(side * 16, 16)],
                                jnp.full((16,), 1.0, jnp.float32))
                sref[pl.ds(r * 16, 16)] = _rsqrt_newton(d)
            return 0
        lax.fori_loop(0, _CH, abstep, 0)
        for side, sref in ((0, sb0), (1, sb1), (2, sb2), (3, sb3)):
            pltpu.sync_copy(
                sref,
                sc_h.at[pl.ds(scb + side * _NPAD * 16 + (base + k * _CH) * 16,
                              _SCW)])
        return 0
    lax.fori_loop(0, _KCH, scprod, 0)
    zdrain(_KCH)
    plsc.subcore_barrier()

    def fetch_scale(sref, side, k):
        pltpu.sync_copy(
            sc_h.at[pl.ds(scb + side * _NPAD * 16 + (base + k * _CH) * 16,
                          _SCW)], sref)

    def scale_to(dst_ref, src_ref, sref):
        def grp(r, _):
            sv = sref[pl.ds(r * 16, 16)]
            for cc in range(_NCOL):
                dst_ref[r, pl.ds(cc * 16, 16)] = src_ref[r, pl.ds(cc * 16, 16)] * sv
            return 0
        lax.fori_loop(0, _CH, grp, 0)

    def prescale_two(src_h, dsta_h, sidea, dstb_h, sideb):
        def pk(k, _):
            pltpu.sync_copy(
                src_h.at[pl.ds(base + k * _CH, _CH), pl.ds(coff, _HALF)], bufa)
            fetch_scale(sb0, sidea, k)
            fetch_scale(sb1, sideb, k)
            scale_to(bufb, bufa, sb0)
            pltpu.sync_copy(bufb, dsta_h.at[pl.ds(toff + base + k * _CH, _CH)])
            scale_to(bufb, bufa, sb1)
            pltpu.sync_copy(bufb, dstb_h.at[pl.ds(toff + base + k * _CH, _CH)])
            return 0
        lax.fori_loop(0, _KCH, pk, 0)

    def scatter_pass(tbl_h):
        C2 = 16
        E2 = _EPT // C2
        NSLOT, PF = 8, 4

        def rq(slot):
            b = bufa if slot < 4 else bufb
            return b.at[pl.ds((slot % 4) * C2, C2)]

        def issue_gather(g, slot):
            iv = svv[pl.ds(g * C2, 16)] + jnp.full((16,), 1, jnp.int32) * toff
            pltpu.async_copy(tbl_h.at[iv], rq(slot), semg.at[slot])

        def wait_gather(slot):
            pltpu.make_async_copy(tbl_h.at[jnp.zeros((16,), jnp.int32)],
                                  rq(slot), semg.at[slot]).wait()

        def issue_scatter(g, slot):
            dv = dvv[pl.ds(g * C2, 16)]
            pltpu.async_copy(rq(slot), acc.at[dv], sems.at[slot], add=True)

        def drain_scatter(slot):
            pltpu.make_async_copy(rq(slot),
                                  acc.at[jnp.zeros((16,), jnp.int32)],
                                  sems.at[slot]).wait()

        for g0 in range(PF):
            issue_gather(g0, g0)
        def oct_(q, _):
            g8 = q * NSLOT
            for par in range(NSLOT):
                g = g8 + par
                nslot = (par + PF) % NSLOT
                wait_gather(par)
                issue_scatter(g, par)
                @pl.when(g + PF < E2)
                def _():
                    @pl.when(g >= NSLOT - PF)
                    def _():
                        drain_scatter(nslot)
                    issue_gather(g + PF, nslot)
            return 0
        lax.fori_loop(0, E2 // NSLOT, oct_, 0)
        for par in range(NSLOT):
            drain_scatter(par)

    def read_acc_scaled(k, side):
        pltpu.sync_copy(acc.at[pl.ds(base + k * _CH, _CH)], bufa)
        zero_acc_chunk(k)
        fetch_scale(sb0, side, k)
        scale_to(bufa, bufa, sb0)

    def add_from(src_h, k, coloff=None):
        if coloff is None:
            pltpu.sync_copy(src_h.at[pl.ds(toff + base + k * _CH, _CH)], bufb)
        else:
            pltpu.sync_copy(
                src_h.at[pl.ds(base + k * _CH, _CH), pl.ds(coloff, _HALF)], bufb)
        def grp(r, _):
            for cc in range(_NCOL):
                bufa[r, pl.ds(cc * 16, 16)] = (
                    bufa[r, pl.ds(cc * 16, 16)] + bufb[r, pl.ds(cc * 16, 16)])
            return 0
        lax.fori_loop(0, _CH, grp, 0)

    def scale_alpha():
        av = jnp.full((16,), _ALPHA, jnp.float32)
        def grp(r, _):
            for cc in range(_NCOL):
                bufa[r, pl.ds(cc * 16, 16)] = bufa[r, pl.ds(cc * 16, 16)] * av
            return 0
        lax.fori_loop(0, _CH, grp, 0)

    prescale_two(e_pos, t0, 0, t1, 2)
    plsc.subcore_barrier()

    stage_idx(sp_h, dp_h)
    scatter_pass(t0)
    plsc.subcore_barrier()
    def wb_l0p(k, _):
        read_acc_scaled(k, 1)
        pltpu.sync_copy(bufa, p1h.at[pl.ds(toff + base + k * _CH, _CH)])
        fetch_scale(sb1, 0, k)
        scale_to(bufb, bufa, sb1)
        pltpu.sync_copy(bufb, t0.at[pl.ds(toff + base + k * _CH, _CH)])
        fetch_scale(sb1, 2, k)
        scale_to(bufb, bufa, sb1)
        pltpu.sync_copy(bufb, t2.at[pl.ds(toff + base + k * _CH, _CH)])
        return 0
    lax.fori_loop(0, _KCH, wb_l0p, 0)
    zdrain(_KCH)
    plsc.subcore_barrier()

    stage_idx(sn_h, dn_h)
    scatter_pass(t1)
    plsc.subcore_barrier()
    def wb_l0n(k, _):
        read_acc_scaled(k, 3)
        pltpu.sync_copy(bufa, n1h.at[pl.ds(toff + base + k * _CH, _CH)])
        fetch_scale(sb1, 2, k)
        scale_to(bufb, bufa, sb1)
        pltpu.sync_copy(bufb, t1.at[pl.ds(toff + base + k * _CH, _CH)])
        fetch_scale(sb1, 0, k)
        scale_to(bufb, bufa, sb1)
        pltpu.sync_copy(bufb, t3.at[pl.ds(toff + base + k * _CH, _CH)])
        return 0
    lax.fori_loop(0, _KCH, wb_l0n, 0)
    zdrain(_KCH)
    plsc.subcore_barrier()

    stage_idx(sp_h, dp_h)
    scatter_pass(t0)
    plsc.subcore_barrier()
    def wb_t(k, _):
        read_acc_scaled(k, 1)
        pltpu.sync_copy(bufa, tth.at[pl.ds(toff + base + k * _CH, _CH)])
        return 0
    lax.fori_loop(0, _KCH, wb_t, 0)
    zdrain(_KCH)
    plsc.subcore_barrier()
    stage_idx(sn_h, dn_h)
    scatter_pass(t1)
    plsc.subcore_barrier()
    def wb_pos(k, _):
        read_acc_scaled(k, 3)
        add_from(tth, k)
        add_from(p1h, k)
        add_from(e_pos, k, coloff=coff)
        scale_alpha()
        pltpu.sync_copy(
            bufa, pos_out.at[pl.ds(base + k * _CH, _CH), pl.ds(coff, _HALF)])
        return 0
    lax.fori_loop(0, _KCH, wb_pos, 0)
    zdrain(_KCH)
    plsc.subcore_barrier()

    stage_idx(sp_h, dp_h)
    scatter_pass(t3)
    plsc.subcore_barrier()
    def wb_t(k, _):
        read_acc_scaled(k, 1)
        pltpu.sync_copy(bufa, tth.at[pl.ds(toff + base + k * _CH, _CH)])
        return 0
    lax.fori_loop(0, _KCH, wb_t, 0)
    zdrain(_KCH)
    plsc.subcore_barrier()
    stage_idx(sn_h, dn_h)
    scatter_pass(t2)
    plsc.subcore_barrier()
    def wb_neg(k, _):
        read_acc_scaled(k, 3)
        add_from(tth, k)
        add_from(n1h, k)
        add_from(e_neg, k, coloff=coff)
        scale_alpha()
        pltpu.sync_copy(
            bufa, neg_out.at[pl.ds(base + k * _CH, _CH), pl.ds(coff, _HALF)])
        return 0
    lax.fori_loop(0, _KCH, wb_neg, 0)
    zdrain(_KCH)


@functools.cache
def _build():
    mesh = plsc.VectorSubcoreMesh(core_axis_name="c", subcore_axis_name="s")
    tbl = pltpu.HBM((_NC * _NPAD, _HALF), jnp.float32)
    return functools.partial(
        pl.kernel,
        mesh=mesh,
        out_type=(jax.ShapeDtypeStruct((_NPAD, _DIM), jnp.float32),
                  jax.ShapeDtypeStruct((_NPAD, _DIM), jnp.float32)),
        scratch_types=[
            tbl, tbl, tbl, tbl, tbl, tbl, tbl,
            pltpu.HBM((_NC * 4 * _NPAD * 16,), jnp.float32),
            pltpu.VMEM((_EPT,), jnp.int32),
            pltpu.VMEM((_EPT,), jnp.int32),
            pltpu.VMEM((_CH, _HALF), jnp.float32),
            pltpu.VMEM((_CH, _HALF), jnp.float32),
            pltpu.VMEM((_SCW,), jnp.float32),
            pltpu.VMEM((_SCW,), jnp.float32),
            pltpu.VMEM((_SCW,), jnp.float32),
            pltpu.VMEM((_SCW,), jnp.float32),
            pltpu.VMEM((8, _HALF), jnp.float32),
            pltpu.VMEM((16, _HALF), jnp.float32),
            pltpu.VMEM_SHARED((_NPAD, _HALF), jnp.float32),
            pltpu.SemaphoreType.DMA((8,)),
            pltpu.SemaphoreType.DMA((8,)),
            pltpu.SemaphoreType.DMA,
        ],
    )(_body)


def _prep_idx(row):
    r = row.astype(jnp.int32).reshape(_NS, _EPW)
    r = jnp.pad(r, ((0, 0), (0, _EPT - _EPW)), constant_values=_NN)
    return r.reshape(_NS * _EPT)


def kernel(user_embedding, item_embedding, user_neg_embedding,
           item_neg_embedding, edge_index_p, edge_index_n):
    e_pos = jnp.concatenate([user_embedding, item_embedding], axis=0)
    e_neg = jnp.concatenate([user_neg_embedding, item_neg_embedding], axis=0)
    e_pos = jnp.pad(e_pos, ((0, _NPAD - _NN), (0, 0)))
    e_neg = jnp.pad(e_neg, ((0, _NPAD - _NN), (0, 0)))
    sp, dp = _prep_idx(edge_index_p[0]), _prep_idx(edge_index_p[1])
    sn, dn = _prep_idx(edge_index_n[0]), _prep_idx(edge_index_n[1])
    pos, neg = _build()(e_pos, e_neg, sp, dp, sn, dn)
    return pos[:_NN], neg[:_NN]


# speedup vs baseline: 4.3682x; 1.0394x over previous
import functools

import jax
import jax.numpy as jnp
from jax import lax
from jax.experimental import pallas as pl
from jax.experimental.pallas import tpu as pltpu
from jax.experimental.pallas import tpu_sc as plsc

_M, _NV, _DIM = 2000, 8000, 256
_NN = _M + _NV
_E = 160000
_NC, _NS = 2, 16
_NPAD = 10240
_RPT = _NPAD // _NS
_CH = 64
_KCH = _RPT // _CH
_EPW = _E // _NS
_EPT = 10240
_ECH = _EPT // _CH
_HALF = 128
_NCOL = _HALF // 16
_SUB = _CH // 16
_SCW = _CH * 16
_ALPHA = 1.0 / 3.0


def _rsqrt_newton(x):
    y = jnp.full((16,), 0.70710678 * 2.0 ** -8, jnp.float32)
    for k in range(7, -1, -1):
        thr = jnp.full((16,), 4.0 ** (k + 1), jnp.float32)
        y = jnp.where(x < thr, jnp.full((16,), 0.70710678 * 2.0 ** -k, jnp.float32), y)
    c15 = jnp.full((16,), 1.5, jnp.float32)
    ch = jnp.full((16,), 0.5, jnp.float32)
    for _ in range(5):
        y = y * (c15 - ch * x * y * y)
    return y


def _body(e_pos, e_neg, sp_h, dp_h, sn_h, dn_h,
          pos_out, neg_out,
          t0, t1, t2, t3, p1h, n1h, tth, sc_h,
          svv, dvv,
          bufa, bufb, bufc,
          sb0, sb1, sb2, sb3,
          zbuf, onesb,
          acc,
          semg, sems, semz):
    cid = lax.axis_index("c")
    sid = lax.axis_index("s")
    base = sid * _RPT
    coff = cid * _HALF
    toff = cid * _NPAD
    scb = (cid * 4) * _NPAD * 16

    def stage_idx(src_hbm, dst_hbm):
        pltpu.sync_copy(src_hbm.at[pl.ds(sid * _EPT, _EPT)], svv)
        pltpu.sync_copy(dst_hbm.at[pl.ds(sid * _EPT, _EPT)], dvv)

    def zfill(r, _):
        zv = jnp.zeros((16,), jnp.float32)
        for cc in range(_NCOL):
            zbuf[r, pl.ds(cc * 16, 16)] = zv
        return 0
    lax.fori_loop(0, 8, zfill, 0)

    def zero_acc_chunk(k):
        for q in range(_CH // 8):
            pltpu.async_copy(zbuf, acc.at[pl.ds(base + k * _CH + q * 8, 8)],
                             semz)

    def zdrain(nchunks):
        def zd(i, _):
            for q in range(_CH // 8):
                pltpu.make_async_copy(zbuf, acc.at[pl.ds(base, 8)], semz).wait()
            return 0
        lax.fori_loop(0, nchunks, zd, 0)

    def zinit(k, _):
        zero_acc_chunk(k)
        return 0
    lax.fori_loop(0, _KCH, zinit, 0)
    zdrain(_KCH)
    plsc.subcore_barrier()

    for rnd, (s_hbm, d_hbm) in ((0, (sp_h, dp_h)), (1, (sn_h, dn_h))):
        stage_idx(s_hbm, d_hbm)
        for half, iv in ((0, svv), (1, dvv)):
            side = rnd * 2 + half
            def bandfill(r, _, side=side):
                for cc in range(_NCOL):
                    v = 1.0 if cc == side else 0.0
                    onesb[r, pl.ds(cc * 16, 16)] = jnp.full((16,), v, jnp.float32)
                return 0
            lax.fori_loop(0, 16, bandfill, 0)
            def ddrain(slot):
                for j in range(_SUB):
                    pltpu.make_async_copy(
                        onesb.at[pl.ds(0, 16)],
                        acc.at[jnp.zeros((16,), jnp.int32)],
                        semg.at[slot]).wait()
            def dquad(q, _, iv=iv):
                g4 = q * 4
                for par in range(4):
                    g = g4 + par
                    @pl.when(g >= 4)
                    def _():
                        ddrain(par)
                    for j in range(_SUB):
                        dv = iv[pl.ds(g * _CH + j * 16, 16)]
                        pltpu.async_copy(onesb.at[pl.ds(0, 16)], acc.at[dv],
                                         semg.at[par], add=True)
                return 0
            lax.fori_loop(0, _ECH // 4, dquad, 0)
            for par in range(4):
                ddrain(par)
    plsc.subcore_barrier()

    def scprod(k, _):
        pltpu.sync_copy(acc.at[pl.ds(base + k * _CH, _CH)], bufa)
        zero_acc_chunk(k)
        def abstep(r, _):
            for side, sref in ((0, sb0), (1, sb1), (2, sb2), (3, sb3)):
                d = jnp.maximum(bufa[r, pl.ds(side * 16, 16)],
                                jnp.full((16,), 1.0, jnp.float32))
                sref[pl.ds(r * 16, 16)] = _rsqrt_newton(d)
            return 0
        lax.fori_loop(0, _CH, abstep, 0)
        for side, sref in ((0, sb0), (1, sb1), (2, sb2), (3, sb3)):
            pltpu.sync_copy(
                sref,
                sc_h.at[pl.ds(scb + side * _NPAD * 16 + (base + k * _CH) * 16,
                              _SCW)])
        return 0
    lax.fori_loop(0, _KCH, scprod, 0)
    zdrain(_KCH)
    plsc.subcore_barrier()

    def fetch_scale(sref, side, k):
        pltpu.sync_copy(
            sc_h.at[pl.ds(scb + side * _NPAD * 16 + (base + k * _CH) * 16,
                          _SCW)], sref)

    def scale_to(dst_ref, src_ref, sref):
        def grp(r, _):
            sv = sref[pl.ds(r * 16, 16)]
            for cc in range(_NCOL):
                dst_ref[r, pl.ds(cc * 16, 16)] = src_ref[r, pl.ds(cc * 16, 16)] * sv
            return 0
        lax.fori_loop(0, _CH, grp, 0)

    def prescale_two(src_h, dsta_h, sidea, dstb_h, sideb):
        def pk(k, _):
            pltpu.sync_copy(
                src_h.at[pl.ds(base + k * _CH, _CH), pl.ds(coff, _HALF)], bufa)
            fetch_scale(sb0, sidea, k)
            fetch_scale(sb1, sideb, k)
            scale_to(bufb, bufa, sb0)
            pltpu.sync_copy(bufb, dsta_h.at[pl.ds(toff + base + k * _CH, _CH)])
            scale_to(bufb, bufa, sb1)
            pltpu.sync_copy(bufb, dstb_h.at[pl.ds(toff + base + k * _CH, _CH)])
            return 0
        lax.fori_loop(0, _KCH, pk, 0)

    def scatter_pass(tbl_h):
        C2 = 16
        E2 = _EPT // C2
        NSLOT, PF = 10, 5

        def rq(slot):
            if slot < 4:
                return bufa.at[pl.ds(slot * C2, C2)]
            if slot < 8:
                return bufb.at[pl.ds((slot - 4) * C2, C2)]
            return bufc.at[pl.ds((slot - 8) * C2, C2)]

        def issue_gather(g, slot):
            iv = svv[pl.ds(g * C2, 16)] + jnp.full((16,), 1, jnp.int32) * toff
            pltpu.async_copy(tbl_h.at[iv], rq(slot), semg.at[slot])

        def wait_gather(slot):
            pltpu.make_async_copy(tbl_h.at[jnp.zeros((16,), jnp.int32)],
                                  rq(slot), semg.at[slot]).wait()

        def issue_scatter(g, slot):
            dv = dvv[pl.ds(g * C2, 16)]
            pltpu.async_copy(rq(slot), acc.at[dv], sems.at[slot], add=True)

        def drain_scatter(slot):
            pltpu.make_async_copy(rq(slot),
                                  acc.at[jnp.zeros((16,), jnp.int32)],
                                  sems.at[slot]).wait()

        for g0 in range(PF):
            issue_gather(g0, g0)
        def ring_(q, _):
            gq = q * NSLOT
            for par in range(NSLOT):
                g = gq + par
                nslot = (par + PF) % NSLOT
                wait_gather(par)
                issue_scatter(g, par)
                @pl.when(g + PF < E2)
                def _():
                    @pl.when(g >= NSLOT - PF)
                    def _():
                        drain_scatter(nslot)
                    issue_gather(g + PF, nslot)
            return 0
        lax.fori_loop(0, E2 // NSLOT, ring_, 0)
        for par in range(NSLOT):
            drain_scatter(par)

    def read_acc_scaled(k, side):
        pltpu.sync_copy(acc.at[pl.ds(base + k * _CH, _CH)], bufa)
        zero_acc_chunk(k)
        fetch_scale(sb0, side, k)
        scale_to(bufa, bufa, sb0)

    def add_from(src_h, k, coloff=None):
        if coloff is None:
            pltpu.sync_copy(src_h.at[pl.ds(toff + base + k * _CH, _CH)], bufb)
        else:
            pltpu.sync_copy(
                src_h.at[pl.ds(base + k * _CH, _CH), pl.ds(coloff, _HALF)], bufb)
        def grp(r, _):
            for cc in range(_NCOL):
                bufa[r, pl.ds(cc * 16, 16)] = (
                    bufa[r, pl.ds(cc * 16, 16)] + bufb[r, pl.ds(cc * 16, 16)])
            return 0
        lax.fori_loop(0, _CH, grp, 0)

    def scale_alpha():
        av = jnp.full((16,), _ALPHA, jnp.float32)
        def grp(r, _):
            for cc in range(_NCOL):
                bufa[r, pl.ds(cc * 16, 16)] = bufa[r, pl.ds(cc * 16, 16)] * av
            return 0
        lax.fori_loop(0, _CH, grp, 0)

    prescale_two(e_pos, t0, 0, t1, 2)
    plsc.subcore_barrier()

    stage_idx(sp_h, dp_h)
    scatter_pass(t0)
    plsc.subcore_barrier()
    def wb_l0p(k, _):
        read_acc_scaled(k, 1)
        pltpu.sync_copy(bufa, p1h.at[pl.ds(toff + base + k * _CH, _CH)])
        fetch_scale(sb1, 0, k)
        scale_to(bufb, bufa, sb1)
        pltpu.sync_copy(bufb, t0.at[pl.ds(toff + base + k * _CH, _CH)])
        fetch_scale(sb1, 2, k)
        scale_to(bufb, bufa, sb1)
        pltpu.sync_copy(bufb, t2.at[pl.ds(toff + base + k * _CH, _CH)])
        return 0
    lax.fori_loop(0, _KCH, wb_l0p, 0)
    zdrain(_KCH)
    plsc.subcore_barrier()

    stage_idx(sn_h, dn_h)
    scatter_pass(t1)
    plsc.subcore_barrier()
    def wb_l0n(k, _):
        read_acc_scaled(k, 3)
        pltpu.sync_copy(bufa, n1h.at[pl.ds(toff + base + k * _CH, _CH)])
        fetch_scale(sb1, 2, k)
        scale_to(bufb, bufa, sb1)
        pltpu.sync_copy(bufb, t1.at[pl.ds(toff + base + k * _CH, _CH)])
        fetch_scale(sb1, 0, k)
        scale_to(bufb, bufa, sb1)
        pltpu.sync_copy(bufb, t3.at[pl.ds(toff + base + k * _CH, _CH)])
        return 0
    lax.fori_loop(0, _KCH, wb_l0n, 0)
    zdrain(_KCH)
    plsc.subcore_barrier()

    stage_idx(sp_h, dp_h)
    scatter_pass(t0)
    plsc.subcore_barrier()
    def wb_t(k, _):
        read_acc_scaled(k, 1)
        pltpu.sync_copy(bufa, tth.at[pl.ds(toff + base + k * _CH, _CH)])
        return 0
    lax.fori_loop(0, _KCH, wb_t, 0)
    zdrain(_KCH)
    plsc.subcore_barrier()
    stage_idx(sn_h, dn_h)
    scatter_pass(t1)
    plsc.subcore_barrier()
    def wb_pos(k, _):
        read_acc_scaled(k, 3)
        add_from(tth, k)
        add_from(p1h, k)
        add_from(e_pos, k, coloff=coff)
        scale_alpha()
        pltpu.sync_copy(
            bufa, pos_out.at[pl.ds(base + k * _CH, _CH), pl.ds(coff, _HALF)])
        return 0
    lax.fori_loop(0, _KCH, wb_pos, 0)
    zdrain(_KCH)
    plsc.subcore_barrier()

    stage_idx(sp_h, dp_h)
    scatter_pass(t3)
    plsc.subcore_barrier()
    def wb_t(k, _):
        read_acc_scaled(k, 1)
        pltpu.sync_copy(bufa, tth.at[pl.ds(toff + base + k * _CH, _CH)])
        return 0
    lax.fori_loop(0, _KCH, wb_t, 0)
    zdrain(_KCH)
    plsc.subcore_barrier()
    stage_idx(sn_h, dn_h)
    scatter_pass(t2)
    plsc.subcore_barrier()
    def wb_neg(k, _):
        read_acc_scaled(k, 3)
        add_from(tth, k)
        add_from(n1h, k)
        add_from(e_neg, k, coloff=coff)
        scale_alpha()
        pltpu.sync_copy(
            bufa, neg_out.at[pl.ds(base + k * _CH, _CH), pl.ds(coff, _HALF)])
        return 0
    lax.fori_loop(0, _KCH, wb_neg, 0)
    zdrain(_KCH)


@functools.cache
def _build():
    mesh = plsc.VectorSubcoreMesh(core_axis_name="c", subcore_axis_name="s")
    tbl = pltpu.HBM((_NC * _NPAD, _HALF), jnp.float32)
    return functools.partial(
        pl.kernel,
        mesh=mesh,
        out_type=(jax.ShapeDtypeStruct((_NPAD, _DIM), jnp.float32),
                  jax.ShapeDtypeStruct((_NPAD, _DIM), jnp.float32)),
        scratch_types=[
            tbl, tbl, tbl, tbl, tbl, tbl, tbl,
            pltpu.HBM((_NC * 4 * _NPAD * 16,), jnp.float32),
            pltpu.VMEM((_EPT,), jnp.int32),
            pltpu.VMEM((_EPT,), jnp.int32),
            pltpu.VMEM((_CH, _HALF), jnp.float32),
            pltpu.VMEM((_CH, _HALF), jnp.float32),
            pltpu.VMEM((32, _HALF), jnp.float32),
            pltpu.VMEM((_SCW,), jnp.float32),
            pltpu.VMEM((_SCW,), jnp.float32),
            pltpu.VMEM((_SCW,), jnp.float32),
            pltpu.VMEM((_SCW,), jnp.float32),
            pltpu.VMEM((8, _HALF), jnp.float32),
            pltpu.VMEM((16, _HALF), jnp.float32),
            pltpu.VMEM_SHARED((_NPAD, _HALF), jnp.float32),
            pltpu.SemaphoreType.DMA((10,)),
            pltpu.SemaphoreType.DMA((10,)),
            pltpu.SemaphoreType.DMA,
        ],
    )(_body)


def _prep_idx(row):
    r = row.astype(jnp.int32).reshape(_NS, _EPW)
    r = jnp.pad(r, ((0, 0), (0, _EPT - _EPW)), constant_values=_NN)
    return r.reshape(_NS * _EPT)


def kernel(user_embedding, item_embedding, user_neg_embedding,
           item_neg_embedding, edge_index_p, edge_index_n):
    e_pos = jnp.concatenate([user_embedding, item_embedding], axis=0)
    e_neg = jnp.concatenate([user_neg_embedding, item_neg_embedding], axis=0)
    e_pos = jnp.pad(e_pos, ((0, _NPAD - _NN), (0, 0)))
    e_neg = jnp.pad(e_neg, ((0, _NPAD - _NN), (0, 0)))
    sp, dp = _prep_idx(edge_index_p[0]), _prep_idx(edge_index_p[1])
    sn, dn = _prep_idx(edge_index_n[0]), _prep_idx(edge_index_n[1])
    pos, neg = _build()(e_pos, e_neg, sp, dp, sn, dn)
    return pos[:_NN], neg[:_NN]
